# baseline (speedup 1.0000x reference)
import functools

import jax
import jax.numpy as jnp
from jax import lax
from jax.experimental import pallas as pl
from jax.experimental.pallas import tpu as pltpu
from jax.experimental.pallas import tpu_sc as plsc

NUM_HEADS = 12
SEQ = 2048
TAB = 512
B = 256
NT = 5
NUM_JOBS = NUM_HEADS * NT


ROWS_TOTAL = NUM_JOBS * B
NW = 32
ROWS_PER_W = ROWS_TOTAL // NW
ROWS_PER_HEAD = NT * B
GW = 1536


CH = 120


def _sc_build_tiles_body(tab_hbm, bt_hbm, tab_v, grev_v, rows_v, sem0, sem1):
    cid = lax.axis_index("c")
    sid = lax.axis_index("s")
    wid = sid * 2 + cid
    r0 = wid * ROWS_PER_W
    h0 = r0 // ROWS_PER_HEAD

    def build_grev(h, slot):
        base = slot * GW
        pltpu.sync_copy(tab_hbm.at[h], tab_v)
        idx511 = jnp.full((16,), TAB - 1, dtype=jnp.int32)
        constv = plsc.load_gather(tab_v, [idx511])
        zerov = jnp.zeros((16,), jnp.float32)
        for c in range(32):
            grev_v[pl.ds(base + 16 * c, 16)] = constv
            grev_v[pl.ds(base + 1024 + 16 * c, 16)] = zerov
        for c in range(32):
            grev_v[pl.ds(base + 512 + 16 * c, 16)] = lax.rev(
                tab_v[pl.ds(TAB - 16 * (c + 1), 16)], (0,)
            )

    build_grev(h0, 0)
    n1 = ROWS_PER_HEAD - (r0 % ROWS_PER_HEAD)

    @pl.when(n1 < ROWS_PER_W)
    def _():
        build_grev(h0 + 1, 1)

    def fill_chunk(q):
        buf = rows_v.at[q % 2]

        def row(k, carry):
            r = r0 + q * CH + k
            rel = r - h0 * ROWS_PER_HEAD
            slot = rel // ROWS_PER_HEAD
            rr = rel - slot * ROWS_PER_HEAD
            t = rr // B
            i = rr - t * B
            s = jnp.where(t >= 4, 1279 - i, 1023 - B * t - i)
            base = slot * GW + s
            for c in range(16):
                buf[k, pl.ds(16 * c, 16)] = grev_v[pl.ds(base + 16 * c, 16)]
            return carry

        lax.fori_loop(0, CH, row, 0)

    sems = [sem0, sem1]
    cps = [None] * 4
    for q in range(4):
        if q >= 2:
            cps[q - 2].wait()
        fill_chunk(q)
        cps[q] = pltpu.async_copy(
            rows_v.at[q % 2], bt_hbm.at[pl.ds(r0 + q * CH, CH)], sems[q % 2]
        )
    cps[2].wait()
    cps[3].wait()


def _build_bias_tiles(table):
    mesh = plsc.VectorSubcoreMesh(core_axis_name="c", subcore_axis_name="s")
    fn = pl.kernel(
        _sc_build_tiles_body,
        mesh=mesh,
        out_type=jax.ShapeDtypeStruct((ROWS_TOTAL, B), jnp.float32),
        scratch_types=[
            pltpu.VMEM((TAB,), jnp.float32),
            pltpu.VMEM((2 * GW,), jnp.float32),
            pltpu.VMEM((2, CH, B), jnp.float32),
            pltpu.SemaphoreType.DMA,
            pltpu.SemaphoreType.DMA,
        ],
        compiler_params=pltpu.CompilerParams(
            needs_layout_passes=False,
            use_tc_tiling_on_sc=False,
        ),
    )
    return fn(table).reshape(NUM_HEADS, NT, B, B)


BQ = 1024
NKB = SEQ // B


def _tc_add_body(attn_ref, bt_ref, out_ref):
    qi = pl.program_id(1)
    for qv in range(SEQ // BQ):

        @pl.when(qi == qv)
        def _():
            for a in range(BQ // B):
                q256 = qv * (BQ // B) + a
                for bcol in range(NKB):
                    d = q256 - bcol
                    t = NT - 1 if d < 0 else min(d, 3)
                    sl = (0, 0, pl.ds(a * B, B), pl.ds(bcol * B, B))
                    out_ref[sl] = attn_ref[sl] + bt_ref[0, t]


def kernel(attn_weights, learnable_bias_diagonals):
    bt = _build_bias_tiles(learnable_bias_diagonals)
    out = pl.pallas_call(
        _tc_add_body,
        grid=(NUM_HEADS, SEQ // BQ),
        in_specs=[
            pl.BlockSpec((1, 1, BQ, SEQ), lambda h, i: (0, h, i, 0)),
            pl.BlockSpec((1, NT, B, B), lambda h, i: (h, 0, 0, 0)),
        ],
        out_specs=pl.BlockSpec((1, 1, BQ, SEQ), lambda h, i: (0, h, i, 0)),
        out_shape=jax.ShapeDtypeStruct(attn_weights.shape, attn_weights.dtype),
        compiler_params=pltpu.CompilerParams(
            dimension_semantics=("parallel", "parallel"),
        ),
    )(attn_weights, bt)
    return out


# speedup vs baseline: 1.1167x; 1.1167x over previous
import jax
import jax.numpy as jnp
from jax import lax
from jax.experimental import pallas as pl
from jax.experimental.pallas import tpu as pltpu
from jax.experimental.pallas import tpu_sc as plsc

NUM_HEADS = 12
SEQ = 2048
TAB = 512
B = 256
NT = 3

ROWS_TOTAL = NUM_HEADS * NT * B
NW = 32
ROWS_PER_W = ROWS_TOTAL // NW
ROWS_PER_HEAD = NT * B
GW = 1536
CH = 96


def _sc_build_tiles_body(tab_hbm, bt_hbm, tab_v, grev_v, rows_v, sem0, sem1):
    cid = lax.axis_index("c")
    sid = lax.axis_index("s")
    wid = sid * 2 + cid
    r0 = wid * ROWS_PER_W
    h0 = r0 // ROWS_PER_HEAD

    def build_grev(h, slot):
        base = slot * GW
        pltpu.sync_copy(tab_hbm.at[h], tab_v)
        idx511 = jnp.full((16,), TAB - 1, dtype=jnp.int32)
        constv = plsc.load_gather(tab_v, [idx511])
        zerov = jnp.zeros((16,), jnp.float32)
        for c in range(32):
            grev_v[pl.ds(base + 16 * c, 16)] = constv
            grev_v[pl.ds(base + 1024 + 16 * c, 16)] = zerov
        for c in range(32):
            grev_v[pl.ds(base + 512 + 16 * c, 16)] = lax.rev(
                tab_v[pl.ds(TAB - 16 * (c + 1), 16)], (0,)
            )

    build_grev(h0, 0)
    n1 = ROWS_PER_HEAD - (r0 % ROWS_PER_HEAD)

    @pl.when(n1 < ROWS_PER_W)
    def _():
        build_grev(h0 + 1, 1)

    def fill_chunk(q):
        buf = rows_v.at[q % 2]

        def row(k, carry):
            r = r0 + q * CH + k
            rel = r - h0 * ROWS_PER_HEAD
            slot = rel // ROWS_PER_HEAD
            rr = rel - slot * ROWS_PER_HEAD
            t = rr // B
            i = rr - t * B
            s = 1023 - B * t - i
            base = slot * GW + s
            for c in range(16):
                buf[k, pl.ds(16 * c, 16)] = grev_v[pl.ds(base + 16 * c, 16)]
            return carry

        lax.fori_loop(0, CH, row, 0)

    sems = [sem0, sem1]
    nch = ROWS_PER_W // CH
    cps = [None] * nch
    for q in range(nch):
        if q >= 2:
            cps[q - 2].wait()
        fill_chunk(q)
        cps[q] = pltpu.async_copy(
            rows_v.at[q % 2], bt_hbm.at[pl.ds(r0 + q * CH, CH)], sems[q % 2]
        )
    for q in range(max(0, nch - 2), nch):
        cps[q].wait()


def _build_bias_tiles(table):
    mesh = plsc.VectorSubcoreMesh(core_axis_name="c", subcore_axis_name="s")
    fn = pl.kernel(
        _sc_build_tiles_body,
        mesh=mesh,
        out_type=jax.ShapeDtypeStruct((ROWS_TOTAL, B), jnp.float32),
        scratch_types=[
            pltpu.VMEM((TAB,), jnp.float32),
            pltpu.VMEM((2 * GW,), jnp.float32),
            pltpu.VMEM((2, CH, B), jnp.float32),
            pltpu.SemaphoreType.DMA,
            pltpu.SemaphoreType.DMA,
        ],
        compiler_params=pltpu.CompilerParams(
            needs_layout_passes=False,
            use_tc_tiling_on_sc=False,
        ),
    )
    return fn(table)


BQ = 1024
NKB = SEQ // B


def _tc_add_body(attn_ref, bt_ref, const_ref, out_ref):
    qi = pl.program_id(1)
    for qv in range(SEQ // BQ):

        @pl.when(qi == qv)
        def _():
            cval = const_ref[0, 0, 0]
            for a in range(BQ // B):
                q256 = qv * (BQ // B) + a
                for bcol in range(NKB):
                    d = q256 - bcol
                    sl = (0, 0, pl.ds(a * B, B), pl.ds(bcol * B, B))
                    if d < 0:
                        out_ref[sl] = attn_ref[sl]
                    elif d < NT:
                        out_ref[sl] = attn_ref[sl] + bt_ref[pl.ds(d * B, B), :]
                    else:
                        out_ref[sl] = attn_ref[sl] + cval


def kernel(attn_weights, learnable_bias_diagonals):
    bt = _build_bias_tiles(learnable_bias_diagonals)
    consts = learnable_bias_diagonals[:, TAB - 1 :].reshape(NUM_HEADS, 1, 1)
    out = pl.pallas_call(
        _tc_add_body,
        grid=(NUM_HEADS, SEQ // BQ),
        in_specs=[
            pl.BlockSpec((1, 1, BQ, SEQ), lambda h, i: (0, h, i, 0)),
            pl.BlockSpec((ROWS_PER_HEAD, B), lambda h, i: (h, 0)),
            pl.BlockSpec((1, 1, 1), lambda h, i: (h, 0, 0), memory_space=pltpu.SMEM),
        ],
        out_specs=pl.BlockSpec((1, 1, BQ, SEQ), lambda h, i: (0, h, i, 0)),
        out_shape=jax.ShapeDtypeStruct(attn_weights.shape, attn_weights.dtype),
        compiler_params=pltpu.CompilerParams(
            dimension_semantics=("parallel", "parallel"),
        ),
    )(attn_weights, bt, consts)
    return out
